# baseline (speedup 1.0000x reference)
import functools
import jax
import jax.numpy as jnp
import numpy as np
from jax import lax
from jax.experimental import pallas as pl
from jax.experimental.pallas import tpu as pltpu
from jax.experimental.pallas import tpu_sc as plsc

IGNORE = 255
H = W = 400
NPIX = H * W
K = NPIX // 4
CHUNK = NPIX // 4
FMAX_BITS = 0x7F800000
NLANE = 16
INNER = CHUNK // NLANE


def _ce_body(pred_ref, tgt_ref, cw_ref, bits_ref, loss_ref, sums_ref):
    c = pl.program_id(0)
    i = pl.program_id(1)
    p0 = pred_ref[0, 0, 0]
    p1 = pred_ref[0, 0, 1]
    t = tgt_ref[0, 0]
    valid = t != IGNORE
    is1 = t == 1
    d = jnp.where(is1, p0 - p1, p1 - p0)
    sp = jnp.maximum(d, 0.0) + jnp.log(1.0 + jnp.exp(-jnp.abs(d)))
    w = jnp.where(is1, cw_ref[c, 1], cw_ref[c, 0])
    loss = jnp.where(valid, w * sp, 0.0)
    bits_ref[0, 0] = lax.bitcast_convert_type(loss, jnp.int32)
    loss_ref[0, 0] = loss
    sums_ref[c, i] = jnp.sum(loss)


_LAST = np.full(NLANE, NLANE - 1, np.int32)


_GDN = lax.GatherDimensionNumbers(
    offset_dims=(), collapsed_slice_dims=(0,), start_index_map=(0,))


def _roll(x, sh):
    idx = (lax.iota(jnp.int32, NLANE) + sh) & (NLANE - 1)
    return lax.gather(x, idx[:, None], _GDN, (1,),
                      mode=lax.GatherScatterMode.PROMISE_IN_BOUNDS)


def _bcast_total(x):
    for sh in (8, 4, 2, 1):
        x = x + _roll(x, sh)
    return x


def _sc_body(bits_hbm, loss_hbm, s_out, c_out, t_out, xchg, vals, vals_f,
             stage_i, stage_f, peers):
    cid = lax.axis_index("c")
    sid = lax.axis_index("s")
    wid = cid * 16 + sid
    rbase = (sid // 4) * 4

    pltpu.sync_copy(bits_hbm.at[pl.ds(wid * CHUNK, CHUNK)], vals)
    pltpu.sync_copy(loss_hbm.at[pl.ds(wid * CHUNK, CHUNK)], vals_f)

    def count_gt(pivot):
        def inner(j, acc):
            v = vals[pl.ds(j * NLANE, NLANE)]
            return acc - ((pivot - v) >> 31)
        return lax.fori_loop(0, INNER, inner, jnp.zeros((NLANE,), jnp.int32))

    def share_count(cnt_lanes):
        stage_i[...] = cnt_lanes
        pltpu.sync_copy(stage_i, xchg.at[wid])
        plsc.subcore_barrier()
        pltpu.sync_copy(xchg.at[pl.ds(cid * 16 + rbase, 4)], peers)
        plsc.subcore_barrier()
        tot_lanes = peers[0] + peers[1] + peers[2] + peers[3]
        return _bcast_total(tot_lanes)

    def step(_, lohi):
        lo, hi = lohi
        mid = lo + ((hi - lo + 1) >> 1)
        tot = share_count(count_gt(mid))
        take = 1 + ((tot - K) >> 31)
        return lo + take * (mid - lo), mid + take * (hi - mid)

    lo0 = jnp.full((NLANE,), -1, jnp.int32)
    hi0 = jnp.full((NLANE,), FMAX_BITS, jnp.int32)
    _, thr = lax.fori_loop(0, 31, step, (lo0, hi0))

    def final(j, carry):
        s, cn = carry
        v = vals[pl.ds(j * NLANE, NLANE)]
        vf = vals_f[pl.ds(j * NLANE, NLANE)]
        gt = -((thr - v) >> 31)
        return s + vf * gt.astype(jnp.float32), cn + gt

    s, cn = lax.fori_loop(0, INNER, final,
                          (jnp.zeros((NLANE,), jnp.float32),
                           jnp.zeros((NLANE,), jnp.int32)))
    stage_f[...] = s
    pltpu.sync_copy(stage_f, s_out.at[wid])
    stage_i[...] = cn
    pltpu.sync_copy(stage_i, c_out.at[wid])
    stage_i[...] = thr
    pltpu.sync_copy(stage_i, t_out.at[wid])


@jax.jit
def kernel(prediction, target, class_weights):
    b = prediction.shape[0]
    pred5 = prediction.reshape(b, 3, 2, H, W)
    bits, loss, sums = pl.pallas_call(
        _ce_body,
        grid=(3, b),
        in_specs=[
            pl.BlockSpec((1, 1, 2, H, W), lambda c, i: (i, c, 0, 0, 0)),
            pl.BlockSpec((1, 1, H, W), lambda c, i: (i, c, 0, 0)),
            pl.BlockSpec(memory_space=pltpu.SMEM),
        ],
        out_specs=[
            pl.BlockSpec((1, 1, H, W), lambda c, i: (c, i, 0, 0)),
            pl.BlockSpec((1, 1, H, W), lambda c, i: (c, i, 0, 0)),
            pl.BlockSpec((3, b), lambda c, i: (0, 0),
                         memory_space=pltpu.SMEM),
        ],
        out_shape=[
            jax.ShapeDtypeStruct((3, b, H, W), jnp.int32),
            jax.ShapeDtypeStruct((3, b, H, W), jnp.float32),
            jax.ShapeDtypeStruct((3, b), jnp.float32),
        ],
    )(pred5, target, class_weights)

    mesh = plsc.VectorSubcoreMesh(core_axis_name="c", subcore_axis_name="s")
    sc = functools.partial(
        pl.kernel,
        out_type=[
            jax.ShapeDtypeStruct((32, NLANE), jnp.float32),
            jax.ShapeDtypeStruct((32, NLANE), jnp.int32),
            jax.ShapeDtypeStruct((32, NLANE), jnp.int32),
            jax.ShapeDtypeStruct((32, NLANE), jnp.int32),
        ],
        mesh=mesh,
        scratch_types=[
            pltpu.VMEM((CHUNK,), jnp.int32),
            pltpu.VMEM((CHUNK,), jnp.float32),
            pltpu.VMEM((NLANE,), jnp.int32),
            pltpu.VMEM((NLANE,), jnp.float32),
            pltpu.VMEM((4, NLANE), jnp.int32),
        ],
    )(_sc_body)
    s_part, c_part, thr, _ = sc(bits.reshape(-1), loss.reshape(-1))

    s4 = s_part.reshape(2, 4, 4, NLANE)
    c4 = c_part.reshape(2, 4, 4, NLANE)
    t4 = thr.reshape(2, 4, 4, NLANE)
    S = jnp.sum(s4, axis=(2, 3))
    C = jnp.sum(c4, axis=(2, 3))
    T = lax.bitcast_convert_type(t4[:, :, 0, 0], jnp.float32)
    topk = S + (K - C).astype(jnp.float32) * T
    total = (jnp.sum(topk[0]) / (b * K)
             + jnp.sum(topk[1]) / (b * K)
             + jnp.sum(sums[2]) / (b * NPIX))
    return total


# speedup vs baseline: 2.0538x; 2.0538x over previous
import functools
import jax
import jax.numpy as jnp
from jax import lax
from jax.experimental import pallas as pl
from jax.experimental.pallas import tpu as pltpu
from jax.experimental.pallas import tpu_sc as plsc

IGNORE = 255
H = W = 400
NPIX = H * W
K = NPIX // 4
CHUNK = NPIX // 4
NLANE = 16
INNER = CHUNK // NLANE
NB = 2048


def _ce_body(pred_ref, tgt_ref, cw_ref, bits_ref, loss_ref, sums_ref):
    c = pl.program_id(0)
    i = pl.program_id(1)
    p0 = pred_ref[0, 0, 0]
    p1 = pred_ref[0, 0, 1]
    t = tgt_ref[0, 0]
    valid = t != IGNORE
    is1 = t == 1
    d = jnp.where(is1, p0 - p1, p1 - p0)
    sp = jnp.maximum(d, 0.0) + jnp.log(1.0 + jnp.exp(-jnp.abs(d)))
    w = jnp.where(is1, cw_ref[c, 1], cw_ref[c, 0])
    loss = jnp.where(valid, w * sp, 0.0)
    bits_ref[0, 0] = lax.bitcast_convert_type(loss, jnp.int32)
    loss_ref[0, 0] = loss
    sums_ref[c, i] = jnp.sum(loss)


_GDN = lax.GatherDimensionNumbers(
    offset_dims=(), collapsed_slice_dims=(0,), start_index_map=(0,))


def _lane_pick(x, lane):
    idx = jnp.full((NLANE,), lane, jnp.int32)
    return lax.gather(x, idx[:, None], _GDN, (1,),
                      mode=lax.GatherScatterMode.PROMISE_IN_BOUNDS)


def _bcast_total(x):
    return _lane_pick(plsc.cumsum(x), NLANE - 1)


def _suffix_incl(x):
    return lax.rev(plsc.cumsum(lax.rev(x, (0,))), (0,))


def _sc_body(bits_hbm, loss_hbm, s_out, c_out, t_out, xc_cnt, xc_sum,
             vals, vals_f, hcnt, hsum, pc, ps, stage_i, stage_f):
    cid = lax.axis_index("c")
    sid = lax.axis_index("s")
    wid = cid * 16 + sid
    rbase = (sid // 4) * 4

    pltpu.sync_copy(bits_hbm.at[pl.ds(wid * CHUNK, CHUNK)], vals)
    pltpu.sync_copy(loss_hbm.at[pl.ds(wid * CHUNK, CHUNK)], vals_f)

    lane = lax.iota(jnp.int32, NLANE)
    ones_i = jnp.full((NLANE,), 1, jnp.int32)
    zeros_i = jnp.zeros((NLANE,), jnp.int32)
    zeros_f = jnp.zeros((NLANE,), jnp.float32)

    def zero_hists(_j, _):
        hcnt[pl.ds(_j * NLANE, NLANE)] = zeros_i
        hsum[pl.ds(_j * NLANE, NLANE)] = zeros_f
        return 0

    def exchange_and_merge():
        pltpu.sync_copy(hcnt, xc_cnt.at[wid])
        pltpu.sync_copy(hsum, xc_sum.at[wid])
        plsc.subcore_barrier()
        pltpu.sync_copy(xc_cnt.at[pl.ds(cid * 16 + rbase, 4)], pc)
        pltpu.sync_copy(xc_sum.at[pl.ds(cid * 16 + rbase, 4)], ps)
        plsc.subcore_barrier()

        def merge(_j, _):
            d = pl.ds(_j * NLANE, NLANE)
            hcnt[d] = pc[0, d] + pc[1, d] + pc[2, d] + pc[3, d]
            hsum[d] = ps[0, d] + ps[1, d] + ps[2, d] + ps[3, d]
            return 0
        lax.fori_loop(0, NB // NLANE, merge, 0)

    def scan_hist(nb, k_cur):
        nbv = nb // NLANE

        def body(_j, carry):
            cum_c, cum_s, b_acc, a_acc, sa_acc = carry
            jj = nbv - 1 - _j
            d = pl.ds(jj * NLANE, NLANE)
            c = hcnt[d]
            s = hsum[d]
            sc_ = _suffix_incl(c)
            ss = _suffix_incl(s)
            incl = cum_c + sc_
            excl = incl - c
            sexcl = cum_s + ss - s
            ind = ((excl < k_cur) & (incl >= k_cur)).astype(jnp.int32)
            indf = ind.astype(jnp.float32)
            b_acc = b_acc + ind * (jj * NLANE + lane)
            a_acc = a_acc + ind * excl
            sa_acc = sa_acc + indf * sexcl
            cum_c = cum_c + _lane_pick(sc_, 0)
            cum_s = cum_s + _lane_pick(ss, 0)
            return cum_c, cum_s, b_acc, a_acc, sa_acc

        _, _, b, a, sa = lax.fori_loop(
            0, nbv, body, (zeros_i, zeros_f, zeros_i, zeros_i, zeros_f))
        return _bcast_total(b), _bcast_total(a), _bcast_total(sa)

    lax.fori_loop(0, NB // NLANE, zero_hists, 0)

    def scat1(_j, _):
        d = pl.ds(_j * NLANE, NLANE)
        v = vals[d]
        idx = v >> 20
        plsc.addupdate_scatter(hcnt, [idx], ones_i)
        plsc.addupdate_scatter(hsum, [idx], vals_f[d])
        return 0
    lax.fori_loop(0, INNER, scat1, 0)
    exchange_and_merge()
    k1 = jnp.full((NLANE,), K, jnp.int32)
    b1, a1, sa1 = scan_hist(NB, k1)
    k2 = k1 - a1

    lax.fori_loop(0, NB // NLANE, zero_hists, 0)

    def scat2(_j, _):
        d = pl.ds(_j * NLANE, NLANE)
        v = vals[d]
        m = (v >> 20) == b1
        idx = (v >> 9) & (NB - 1)
        plsc.addupdate_scatter(hcnt, [idx], ones_i, mask=m)
        plsc.addupdate_scatter(hsum, [idx], vals_f[d], mask=m)
        return 0
    lax.fori_loop(0, INNER, scat2, 0)
    exchange_and_merge()
    b2, a2, sa2 = scan_hist(NB, k2)
    k3 = k2 - a2

    lax.fori_loop(0, NB // NLANE, zero_hists, 0)
    pref = b1 * 2048 + b2

    def scat3(_j, _):
        d = pl.ds(_j * NLANE, NLANE)
        v = vals[d]
        m = (v >> 9) == pref
        idx = v & 511
        plsc.addupdate_scatter(hcnt, [idx], ones_i, mask=m)
        plsc.addupdate_scatter(hsum, [idx], vals_f[d], mask=m)
        return 0
    lax.fori_loop(0, INNER, scat3, 0)
    exchange_and_merge()
    b3, a3, sa3 = scan_hist(512, k3)
    k4 = k3 - a3

    thr = b1 * 1048576 + b2 * 512 + b3
    s_above = sa1 + sa2 + sa3

    stage_i[...] = thr
    pltpu.sync_copy(stage_i, t_out.at[wid])
    stage_i[...] = k4
    pltpu.sync_copy(stage_i, c_out.at[wid])
    stage_f[...] = s_above
    pltpu.sync_copy(stage_f, s_out.at[wid])


@jax.jit
def kernel(prediction, target, class_weights):
    b = prediction.shape[0]
    pred5 = prediction.reshape(b, 3, 2, H, W)
    bits, loss, sums = pl.pallas_call(
        _ce_body,
        grid=(3, b),
        in_specs=[
            pl.BlockSpec((1, 1, 2, H, W), lambda c, i: (i, c, 0, 0, 0)),
            pl.BlockSpec((1, 1, H, W), lambda c, i: (i, c, 0, 0)),
            pl.BlockSpec(memory_space=pltpu.SMEM),
        ],
        out_specs=[
            pl.BlockSpec((1, 1, H, W), lambda c, i: (c, i, 0, 0)),
            pl.BlockSpec((1, 1, H, W), lambda c, i: (c, i, 0, 0)),
            pl.BlockSpec((3, b), lambda c, i: (0, 0),
                         memory_space=pltpu.SMEM),
        ],
        out_shape=[
            jax.ShapeDtypeStruct((3, b, H, W), jnp.int32),
            jax.ShapeDtypeStruct((3, b, H, W), jnp.float32),
            jax.ShapeDtypeStruct((3, b), jnp.float32),
        ],
    )(pred5, target, class_weights)

    mesh = plsc.VectorSubcoreMesh(core_axis_name="c", subcore_axis_name="s")
    sc = functools.partial(
        pl.kernel,
        out_type=[
            jax.ShapeDtypeStruct((32, NLANE), jnp.float32),
            jax.ShapeDtypeStruct((32, NLANE), jnp.int32),
            jax.ShapeDtypeStruct((32, NLANE), jnp.int32),
            jax.ShapeDtypeStruct((32, NB), jnp.int32),
            jax.ShapeDtypeStruct((32, NB), jnp.float32),
        ],
        mesh=mesh,
        compiler_params=pltpu.CompilerParams(needs_layout_passes=False),
        scratch_types=[
            pltpu.VMEM((CHUNK,), jnp.int32),
            pltpu.VMEM((CHUNK,), jnp.float32),
            pltpu.VMEM((NB,), jnp.int32),
            pltpu.VMEM((NB,), jnp.float32),
            pltpu.VMEM((4, NB), jnp.int32),
            pltpu.VMEM((4, NB), jnp.float32),
            pltpu.VMEM((NLANE,), jnp.int32),
            pltpu.VMEM((NLANE,), jnp.float32),
        ],
    )(_sc_body)
    s_part, k4_part, thr_part, _, _ = sc(bits.reshape(-1), loss.reshape(-1))

    S = s_part.reshape(8, 4, NLANE)[:, 0, 0]
    K4 = k4_part.reshape(8, 4, NLANE)[:, 0, 0].astype(jnp.float32)
    T = lax.bitcast_convert_type(thr_part.reshape(8, 4, NLANE)[:, 0, 0],
                                 jnp.float32)
    topk = (S + K4 * T).reshape(2, 4)
    total = (jnp.sum(topk[0]) / (b * K)
             + jnp.sum(topk[1]) / (b * K)
             + jnp.sum(sums[2]) / (b * NPIX))
    return total


# speedup vs baseline: 2.7202x; 1.3244x over previous
import functools
import jax
import jax.numpy as jnp
from jax import lax
from jax.experimental import pallas as pl
from jax.experimental.pallas import tpu as pltpu
from jax.experimental.pallas import tpu_sc as plsc

IGNORE = 255
H = W = 400
NPIX = H * W
K = NPIX // 4
CHUNK = NPIX // 4
NLANE = 16
INNER = CHUNK // NLANE
NB = 2048


def _ce_body(pred_ref, tgt_ref, cw_ref, bits_ref, sums_ref):
    c = pl.program_id(0)
    i = pl.program_id(1)
    p0 = pred_ref[0, 0, 0]
    p1 = pred_ref[0, 0, 1]
    t = tgt_ref[0, 0]
    valid = t != IGNORE
    is1 = t == 1
    d = jnp.where(is1, p0 - p1, p1 - p0)
    sp = jnp.maximum(d, 0.0) + jnp.log(1.0 + jnp.exp(-jnp.abs(d)))
    w = jnp.where(is1, cw_ref[c, 1], cw_ref[c, 0])
    loss = jnp.where(valid, w * sp, 0.0)
    bits_ref[0, 0] = lax.bitcast_convert_type(loss, jnp.int32)
    sums_ref[c, i] = jnp.sum(loss)


_GDN = lax.GatherDimensionNumbers(
    offset_dims=(), collapsed_slice_dims=(0,), start_index_map=(0,))


def _lane_pick(x, lane):
    idx = jnp.full((NLANE,), lane, jnp.int32)
    return lax.gather(x, idx[:, None], _GDN, (1,),
                      mode=lax.GatherScatterMode.PROMISE_IN_BOUNDS)


def _bcast_total(x):
    return _lane_pick(plsc.cumsum(x), NLANE - 1)


def _suffix_incl(x):
    return lax.rev(plsc.cumsum(lax.rev(x, (0,))), (0,))


def _sc_body(bits_hbm, s_out, c_out, t_out, xc_cnt, xc_sum,
             vals, hcnt, hsum, pc, ps, stage_i, stage_f):
    cid = lax.axis_index("c")
    sid = lax.axis_index("s")
    wid = cid * 16 + sid
    rbase = (sid // 4) * 4

    pltpu.sync_copy(bits_hbm.at[pl.ds(wid * CHUNK, CHUNK)], vals)

    lane = lax.iota(jnp.int32, NLANE)
    ones_i = jnp.full((NLANE,), 1, jnp.int32)
    zeros_i = jnp.zeros((NLANE,), jnp.int32)
    zeros_f = jnp.zeros((NLANE,), jnp.float32)

    def zero_hists(_j, _):
        hcnt[pl.ds(_j * NLANE, NLANE)] = zeros_i
        hsum[pl.ds(_j * NLANE, NLANE)] = zeros_f
        return 0

    def exchange_and_merge():
        pltpu.sync_copy(hcnt, xc_cnt.at[wid])
        pltpu.sync_copy(hsum, xc_sum.at[wid])
        plsc.subcore_barrier()
        pltpu.sync_copy(xc_cnt.at[pl.ds(cid * 16 + rbase, 4)], pc)
        pltpu.sync_copy(xc_sum.at[pl.ds(cid * 16 + rbase, 4)], ps)
        plsc.subcore_barrier()

        def merge(_j, _):
            d = pl.ds(_j * NLANE, NLANE)
            hcnt[d] = pc[0, d] + pc[1, d] + pc[2, d] + pc[3, d]
            hsum[d] = ps[0, d] + ps[1, d] + ps[2, d] + ps[3, d]
            return 0
        lax.fori_loop(0, NB // NLANE, merge, 0, unroll=4)

    def scan_hist(nb, k_cur):
        nbv = nb // NLANE

        def body(_j, carry):
            cum_c, cum_s, b_acc, a_acc, sa_acc = carry
            jj = nbv - 1 - _j
            d = pl.ds(jj * NLANE, NLANE)
            c = hcnt[d]
            s = hsum[d]
            sc_ = _suffix_incl(c)
            ss = _suffix_incl(s)
            incl = cum_c + sc_
            excl = incl - c
            sexcl = cum_s + ss - s
            ind = ((excl < k_cur) & (incl >= k_cur)).astype(jnp.int32)
            indf = ind.astype(jnp.float32)
            b_acc = b_acc + ind * (jj * NLANE + lane)
            a_acc = a_acc + ind * excl
            sa_acc = sa_acc + indf * sexcl
            cum_c = cum_c + _lane_pick(sc_, 0)
            cum_s = cum_s + _lane_pick(ss, 0)
            return cum_c, cum_s, b_acc, a_acc, sa_acc

        _, _, b, a, sa = lax.fori_loop(
            0, nbv, body, (zeros_i, zeros_f, zeros_i, zeros_i, zeros_f))
        return _bcast_total(b), _bcast_total(a), _bcast_total(sa)

    lax.fori_loop(0, NB // NLANE, zero_hists, 0, unroll=4)

    def scat1(_j, _):
        d = pl.ds(_j * NLANE, NLANE)
        v = vals[d]
        idx = v >> 20
        plsc.addupdate_scatter(hcnt, [idx], ones_i)
        plsc.addupdate_scatter(hsum, [idx], plsc.bitcast(v, jnp.float32))
        return 0
    lax.fori_loop(0, INNER, scat1, 0, unroll=8)
    exchange_and_merge()
    k1 = jnp.full((NLANE,), K, jnp.int32)
    b1, a1, sa1 = scan_hist(NB, k1)
    k2 = k1 - a1

    lax.fori_loop(0, NB // NLANE, zero_hists, 0, unroll=4)

    def scat2(_j, _):
        d = pl.ds(_j * NLANE, NLANE)
        v = vals[d]
        m = (v >> 20) == b1
        idx = (v >> 9) & (NB - 1)
        plsc.addupdate_scatter(hcnt, [idx], ones_i, mask=m)
        plsc.addupdate_scatter(hsum, [idx], plsc.bitcast(v, jnp.float32),
                               mask=m)
        return 0
    lax.fori_loop(0, INNER, scat2, 0, unroll=8)
    exchange_and_merge()
    b2, a2, sa2 = scan_hist(NB, k2)
    k3 = k2 - a2

    lax.fori_loop(0, NB // NLANE, zero_hists, 0, unroll=4)
    pref = b1 * 2048 + b2

    def scat3(_j, _):
        d = pl.ds(_j * NLANE, NLANE)
        v = vals[d]
        m = (v >> 9) == pref
        idx = v & 511
        plsc.addupdate_scatter(hcnt, [idx], ones_i, mask=m)
        plsc.addupdate_scatter(hsum, [idx], plsc.bitcast(v, jnp.float32),
                               mask=m)
        return 0
    lax.fori_loop(0, INNER, scat3, 0, unroll=8)
    exchange_and_merge()
    b3, a3, sa3 = scan_hist(512, k3)
    k4 = k3 - a3

    thr = b1 * 1048576 + b2 * 512 + b3
    s_above = sa1 + sa2 + sa3

    stage_i[...] = thr
    pltpu.sync_copy(stage_i, t_out.at[wid])
    stage_i[...] = k4
    pltpu.sync_copy(stage_i, c_out.at[wid])
    stage_f[...] = s_above
    pltpu.sync_copy(stage_f, s_out.at[wid])


@jax.jit
def kernel(prediction, target, class_weights):
    b = prediction.shape[0]
    pred5 = prediction.reshape(b, 3, 2, H, W)
    bits, sums = pl.pallas_call(
        _ce_body,
        grid=(3, b),
        in_specs=[
            pl.BlockSpec((1, 1, 2, H, W), lambda c, i: (i, c, 0, 0, 0)),
            pl.BlockSpec((1, 1, H, W), lambda c, i: (i, c, 0, 0)),
            pl.BlockSpec(memory_space=pltpu.SMEM),
        ],
        out_specs=[
            pl.BlockSpec((1, 1, H, W), lambda c, i: (c, i, 0, 0)),
            pl.BlockSpec((3, b), lambda c, i: (0, 0),
                         memory_space=pltpu.SMEM),
        ],
        out_shape=[
            jax.ShapeDtypeStruct((3, b, H, W), jnp.int32),
            jax.ShapeDtypeStruct((3, b), jnp.float32),
        ],
    )(pred5, target, class_weights)

    mesh = plsc.VectorSubcoreMesh(core_axis_name="c", subcore_axis_name="s")
    sc = functools.partial(
        pl.kernel,
        out_type=[
            jax.ShapeDtypeStruct((32, NLANE), jnp.float32),
            jax.ShapeDtypeStruct((32, NLANE), jnp.int32),
            jax.ShapeDtypeStruct((32, NLANE), jnp.int32),
            jax.ShapeDtypeStruct((32, NB), jnp.int32),
            jax.ShapeDtypeStruct((32, NB), jnp.float32),
        ],
        mesh=mesh,
        compiler_params=pltpu.CompilerParams(needs_layout_passes=False),
        scratch_types=[
            pltpu.VMEM((CHUNK,), jnp.int32),
            pltpu.VMEM((NB,), jnp.int32),
            pltpu.VMEM((NB,), jnp.float32),
            pltpu.VMEM((4, NB), jnp.int32),
            pltpu.VMEM((4, NB), jnp.float32),
            pltpu.VMEM((NLANE,), jnp.int32),
            pltpu.VMEM((NLANE,), jnp.float32),
        ],
    )(_sc_body)
    s_part, k4_part, thr_part, _, _ = sc(bits.reshape(-1))

    S = s_part.reshape(8, 4, NLANE)[:, 0, 0]
    K4 = k4_part.reshape(8, 4, NLANE)[:, 0, 0].astype(jnp.float32)
    T = lax.bitcast_convert_type(thr_part.reshape(8, 4, NLANE)[:, 0, 0],
                                 jnp.float32)
    topk = (S + K4 * T).reshape(2, 4)
    total = (jnp.sum(topk[0]) / (b * K)
             + jnp.sum(topk[1]) / (b * K)
             + jnp.sum(sums[2]) / (b * NPIX))
    return total


# speedup vs baseline: 2.9411x; 1.0812x over previous
import functools
import jax
import jax.numpy as jnp
from jax import lax
from jax.experimental import pallas as pl
from jax.experimental.pallas import tpu as pltpu
from jax.experimental.pallas import tpu_sc as plsc

IGNORE = 255
H = W = 400
NPIX = H * W
K = NPIX // 4
CHUNK = NPIX // 4
NLANE = 16
INNER = CHUNK // NLANE
NB = 2048


def _ce_body(pred_ref, tgt_ref, cw_ref, bits_ref, sums_ref):
    c = pl.program_id(0)
    i = pl.program_id(1)
    p0 = pred_ref[0, 0, 0]
    p1 = pred_ref[0, 0, 1]
    t = tgt_ref[0, 0]
    valid = t != IGNORE
    is1 = t == 1
    d = jnp.where(is1, p0 - p1, p1 - p0)
    sp = jnp.maximum(d, 0.0) + jnp.log(1.0 + jnp.exp(-jnp.abs(d)))
    w = jnp.where(is1, cw_ref[c, 1], cw_ref[c, 0])
    loss = jnp.where(valid, w * sp, 0.0)
    bits_ref[0, 0] = lax.bitcast_convert_type(loss, jnp.int32)
    sums_ref[c, i] = jnp.sum(loss)


_GDN = lax.GatherDimensionNumbers(
    offset_dims=(), collapsed_slice_dims=(0,), start_index_map=(0,))


def _lane_pick(x, lane):
    idx = jnp.full((NLANE,), lane, jnp.int32)
    return lax.gather(x, idx[:, None], _GDN, (1,),
                      mode=lax.GatherScatterMode.PROMISE_IN_BOUNDS)


def _bcast_total(x):
    return _lane_pick(plsc.cumsum(x), NLANE - 1)


def _suffix_incl(x):
    return lax.rev(plsc.cumsum(lax.rev(x, (0,))), (0,))


def _sc_body(bits_hbm, s_out, c_out, t_out, xc_cnt, xc_sum,
             vals, hcnt, pc, psf, stage_i, stage_f):
    cid = lax.axis_index("c")
    sid = lax.axis_index("s")
    wid = cid * 16 + sid
    rbase = (sid // 4) * 4

    pltpu.sync_copy(bits_hbm.at[pl.ds(wid * CHUNK, CHUNK)], vals)

    lane = lax.iota(jnp.int32, NLANE)
    ones_i = jnp.full((NLANE,), 1, jnp.int32)
    zeros_i = jnp.zeros((NLANE,), jnp.int32)
    zeros_f = jnp.zeros((NLANE,), jnp.float32)

    def zero_hists(_j, _):
        hcnt[pl.ds(_j * NLANE, NLANE)] = zeros_i
        return 0

    def exchange_and_merge():
        pltpu.sync_copy(hcnt, xc_cnt.at[wid])
        plsc.subcore_barrier()
        pltpu.sync_copy(xc_cnt.at[pl.ds(cid * 16 + rbase, 4)], pc)
        plsc.subcore_barrier()

        def merge(_j, _):
            d = pl.ds(_j * NLANE, NLANE)
            hcnt[d] = pc[0, d] + pc[1, d] + pc[2, d] + pc[3, d]
            return 0
        lax.fori_loop(0, NB // NLANE, merge, 0, unroll=4)

    def scan_hist(nb, k_cur):
        nbv = nb // NLANE

        def body(_j, carry):
            cum_c, b_acc, a_acc = carry
            jj = nbv - 1 - _j
            d = pl.ds(jj * NLANE, NLANE)
            c = hcnt[d]
            sc_ = _suffix_incl(c)
            incl = cum_c + sc_
            excl = incl - c
            ind = ((excl < k_cur) & (incl >= k_cur)).astype(jnp.int32)
            b_acc = b_acc + ind * (jj * NLANE + lane)
            a_acc = a_acc + ind * excl
            cum_c = cum_c + _lane_pick(sc_, 0)
            return cum_c, b_acc, a_acc

        _, b, a = lax.fori_loop(
            0, nbv, body, (zeros_i, zeros_i, zeros_i))
        return _bcast_total(b), _bcast_total(a)

    lax.fori_loop(0, NB // NLANE, zero_hists, 0, unroll=4)

    def scat1(_j, _):
        d = pl.ds(_j * NLANE, NLANE)
        v = vals[d]
        idx = v >> 20
        plsc.addupdate_scatter(hcnt, [idx], ones_i)
        return 0
    lax.fori_loop(0, INNER, scat1, 0, unroll=8)
    exchange_and_merge()
    k1 = jnp.full((NLANE,), K, jnp.int32)
    b1, a1 = scan_hist(NB, k1)
    k2 = k1 - a1

    lax.fori_loop(0, NB // NLANE, zero_hists, 0, unroll=4)

    def scat2(_j, _):
        d = pl.ds(_j * NLANE, NLANE)
        v = vals[d]
        m = (v >> 20) == b1
        idx = (v >> 9) & (NB - 1)
        plsc.addupdate_scatter(hcnt, [idx], ones_i, mask=m)
        return 0
    lax.fori_loop(0, INNER, scat2, 0, unroll=8)
    exchange_and_merge()
    b2, a2 = scan_hist(NB, k2)
    k3 = k2 - a2

    lax.fori_loop(0, NB // NLANE, zero_hists, 0, unroll=4)
    pref = b1 * 2048 + b2

    def scat3(_j, _):
        d = pl.ds(_j * NLANE, NLANE)
        v = vals[d]
        m = (v >> 9) == pref
        idx = v & 511
        plsc.addupdate_scatter(hcnt, [idx], ones_i, mask=m)
        return 0
    lax.fori_loop(0, INNER, scat3, 0, unroll=8)
    exchange_and_merge()
    b3, a3 = scan_hist(512, k3)
    k4 = k3 - a3

    thr = b1 * 1048576 + b2 * 512 + b3

    def fsum(_j, acc):
        v = vals[pl.ds(_j * NLANE, NLANE)]
        return acc + jnp.where(v > thr, plsc.bitcast(v, jnp.float32), 0.0)
    s_loc = lax.fori_loop(0, INNER, fsum, zeros_f, unroll=8)
    stage_f[...] = s_loc
    pltpu.sync_copy(stage_f, xc_sum.at[wid])
    plsc.subcore_barrier()
    pltpu.sync_copy(xc_sum.at[pl.ds(cid * 16 + rbase, 4)], psf)
    plsc.subcore_barrier()
    s_above = _bcast_total(psf[0] + psf[1] + psf[2] + psf[3])

    stage_i[...] = thr
    pltpu.sync_copy(stage_i, t_out.at[wid])
    stage_i[...] = k4
    pltpu.sync_copy(stage_i, c_out.at[wid])
    stage_f[...] = s_above
    pltpu.sync_copy(stage_f, s_out.at[wid])


@jax.jit
def kernel(prediction, target, class_weights):
    b = prediction.shape[0]
    pred5 = prediction.reshape(b, 3, 2, H, W)
    bits, sums = pl.pallas_call(
        _ce_body,
        grid=(3, b),
        in_specs=[
            pl.BlockSpec((1, 1, 2, H, W), lambda c, i: (i, c, 0, 0, 0)),
            pl.BlockSpec((1, 1, H, W), lambda c, i: (i, c, 0, 0)),
            pl.BlockSpec(memory_space=pltpu.SMEM),
        ],
        out_specs=[
            pl.BlockSpec((1, 1, H, W), lambda c, i: (c, i, 0, 0)),
            pl.BlockSpec((3, b), lambda c, i: (0, 0),
                         memory_space=pltpu.SMEM),
        ],
        out_shape=[
            jax.ShapeDtypeStruct((3, b, H, W), jnp.int32),
            jax.ShapeDtypeStruct((3, b), jnp.float32),
        ],
    )(pred5, target, class_weights)

    mesh = plsc.VectorSubcoreMesh(core_axis_name="c", subcore_axis_name="s")
    sc = functools.partial(
        pl.kernel,
        out_type=[
            jax.ShapeDtypeStruct((32, NLANE), jnp.float32),
            jax.ShapeDtypeStruct((32, NLANE), jnp.int32),
            jax.ShapeDtypeStruct((32, NLANE), jnp.int32),
            jax.ShapeDtypeStruct((32, NB), jnp.int32),
            jax.ShapeDtypeStruct((32, NLANE), jnp.float32),
        ],
        mesh=mesh,
        compiler_params=pltpu.CompilerParams(needs_layout_passes=False),
        scratch_types=[
            pltpu.VMEM((CHUNK,), jnp.int32),
            pltpu.VMEM((NB,), jnp.int32),
            pltpu.VMEM((4, NB), jnp.int32),
            pltpu.VMEM((4, NLANE), jnp.float32),
            pltpu.VMEM((NLANE,), jnp.int32),
            pltpu.VMEM((NLANE,), jnp.float32),
        ],
    )(_sc_body)
    s_part, k4_part, thr_part, _, _ = sc(bits.reshape(-1))

    S = s_part.reshape(8, 4, NLANE)[:, 0, 0]
    K4 = k4_part.reshape(8, 4, NLANE)[:, 0, 0].astype(jnp.float32)
    T = lax.bitcast_convert_type(thr_part.reshape(8, 4, NLANE)[:, 0, 0],
                                 jnp.float32)
    topk = (S + K4 * T).reshape(2, 4)
    total = (jnp.sum(topk[0]) / (b * K)
             + jnp.sum(topk[1]) / (b * K)
             + jnp.sum(sums[2]) / (b * NPIX))
    return total


# speedup vs baseline: 3.2175x; 1.0940x over previous
import functools
import jax
import jax.numpy as jnp
from jax import lax
from jax.experimental import pallas as pl
from jax.experimental.pallas import tpu as pltpu
from jax.experimental.pallas import tpu_sc as plsc

IGNORE = 255
H = W = 400
NPIX = H * W
K = NPIX // 4
CHUNK = NPIX // 4
NLANE = 16
INNER = CHUNK // NLANE
NB = 2048


def _loss_plane(pred_ref, tgt_ref, cw_ref, c):
    p0 = pred_ref[0, 0, 0]
    p1 = pred_ref[0, 0, 1]
    t = tgt_ref[0, 0]
    valid = t != IGNORE
    is1 = t == 1
    d = jnp.where(is1, p0 - p1, p1 - p0)
    sp = jnp.maximum(d, 0.0) + jnp.log(1.0 + jnp.exp(-jnp.abs(d)))
    w = jnp.where(is1, cw_ref[c, 1], cw_ref[c, 0])
    return jnp.where(valid, w * sp, 0.0)


def _ce_body(pred_ref, tgt_ref, cw_ref, bits_ref):
    loss = _loss_plane(pred_ref, tgt_ref, cw_ref, pl.program_id(0))
    bits_ref[0, 0] = lax.bitcast_convert_type(loss, jnp.int32)


def _ce2_body(pred_ref, tgt_ref, cw_ref, sums_ref):
    i = pl.program_id(0)
    loss = _loss_plane(pred_ref, tgt_ref, cw_ref, 2)
    sums_ref[0, i] = jnp.sum(loss)


_GDN = lax.GatherDimensionNumbers(
    offset_dims=(), collapsed_slice_dims=(0,), start_index_map=(0,))


def _lane_pick(x, lane):
    idx = jnp.full((NLANE,), lane, jnp.int32)
    return lax.gather(x, idx[:, None], _GDN, (1,),
                      mode=lax.GatherScatterMode.PROMISE_IN_BOUNDS)


def _bcast_total(x):
    return _lane_pick(plsc.cumsum(x), NLANE - 1)


def _suffix_incl(x):
    return lax.rev(plsc.cumsum(lax.rev(x, (0,))), (0,))


def _sc_body(bits_hbm, s_out, c_out, t_out, xc_cnt, xc_sum,
             vals, hcnt, pc, psf, stage_i, stage_f):
    cid = lax.axis_index("c")
    sid = lax.axis_index("s")
    wid = cid * 16 + sid
    rbase = (sid // 4) * 4

    pltpu.sync_copy(bits_hbm.at[pl.ds(wid * CHUNK, CHUNK)], vals)

    lane = lax.iota(jnp.int32, NLANE)
    ones_i = jnp.full((NLANE,), 1, jnp.int32)
    zeros_i = jnp.zeros((NLANE,), jnp.int32)
    zeros_f = jnp.zeros((NLANE,), jnp.float32)

    def zero_hists(_j, _):
        hcnt[pl.ds(_j * NLANE, NLANE)] = zeros_i
        return 0

    def exchange_and_merge():
        pltpu.sync_copy(hcnt, xc_cnt.at[wid])
        plsc.subcore_barrier()
        pltpu.sync_copy(xc_cnt.at[pl.ds(cid * 16 + rbase, 4)], pc)
        plsc.subcore_barrier()

        def merge(_j, _):
            d = pl.ds(_j * NLANE, NLANE)
            hcnt[d] = pc[0, d] + pc[1, d] + pc[2, d] + pc[3, d]
            return 0
        lax.fori_loop(0, NB // NLANE, merge, 0, unroll=4)

    def scan_hist(nb, k_cur):
        nbv = nb // NLANE

        def body(_j, carry):
            cum_c, b_acc, a_acc = carry
            jj = nbv - 1 - _j
            d = pl.ds(jj * NLANE, NLANE)
            c = hcnt[d]
            sc_ = _suffix_incl(c)
            incl = cum_c + sc_
            excl = incl - c
            ind = ((excl < k_cur) & (incl >= k_cur)).astype(jnp.int32)
            b_acc = b_acc + ind * (jj * NLANE + lane)
            a_acc = a_acc + ind * excl
            cum_c = cum_c + _lane_pick(sc_, 0)
            return cum_c, b_acc, a_acc

        _, b, a = lax.fori_loop(
            0, nbv, body, (zeros_i, zeros_i, zeros_i))
        return _bcast_total(b), _bcast_total(a)

    lax.fori_loop(0, NB // NLANE, zero_hists, 0, unroll=4)

    def scat1(_j, _):
        d = pl.ds(_j * NLANE, NLANE)
        v = vals[d]
        idx = v >> 20
        plsc.addupdate_scatter(hcnt, [idx], ones_i)
        return 0
    lax.fori_loop(0, INNER, scat1, 0, unroll=8)
    exchange_and_merge()
    k1 = jnp.full((NLANE,), K, jnp.int32)
    b1, a1 = scan_hist(NB, k1)
    k2 = k1 - a1

    lax.fori_loop(0, NB // NLANE, zero_hists, 0, unroll=4)

    def scat2(_j, _):
        d = pl.ds(_j * NLANE, NLANE)
        v = vals[d]
        m = (v >> 20) == b1
        idx = (v >> 9) & (NB - 1)
        plsc.addupdate_scatter(hcnt, [idx], ones_i, mask=m)
        return 0
    lax.fori_loop(0, INNER, scat2, 0, unroll=8)
    exchange_and_merge()
    b2, a2 = scan_hist(NB, k2)
    k3 = k2 - a2

    lax.fori_loop(0, NB // NLANE, zero_hists, 0, unroll=4)
    pref = b1 * 2048 + b2

    def scat3(_j, _):
        d = pl.ds(_j * NLANE, NLANE)
        v = vals[d]
        m = (v >> 9) == pref
        idx = v & 511
        plsc.addupdate_scatter(hcnt, [idx], ones_i, mask=m)
        return 0
    lax.fori_loop(0, INNER, scat3, 0, unroll=8)
    exchange_and_merge()
    b3, a3 = scan_hist(512, k3)
    k4 = k3 - a3

    thr = b1 * 1048576 + b2 * 512 + b3

    def fsum(_j, acc):
        v = vals[pl.ds(_j * NLANE, NLANE)]
        return acc + jnp.where(v > thr, plsc.bitcast(v, jnp.float32), 0.0)
    s_loc = lax.fori_loop(0, INNER, fsum, zeros_f, unroll=8)
    stage_f[...] = s_loc
    pltpu.sync_copy(stage_f, xc_sum.at[wid])
    plsc.subcore_barrier()
    pltpu.sync_copy(xc_sum.at[pl.ds(cid * 16 + rbase, 4)], psf)
    plsc.subcore_barrier()
    s_above = _bcast_total(psf[0] + psf[1] + psf[2] + psf[3])

    stage_i[...] = thr
    pltpu.sync_copy(stage_i, t_out.at[wid])
    stage_i[...] = k4
    pltpu.sync_copy(stage_i, c_out.at[wid])
    stage_f[...] = s_above
    pltpu.sync_copy(stage_f, s_out.at[wid])


@jax.jit
def kernel(prediction, target, class_weights):
    b = prediction.shape[0]
    pred5 = prediction.reshape(b, 3, 2, H, W)
    bits = pl.pallas_call(
        _ce_body,
        grid=(2, b),
        in_specs=[
            pl.BlockSpec((1, 1, 2, H, W), lambda c, i: (i, c, 0, 0, 0)),
            pl.BlockSpec((1, 1, H, W), lambda c, i: (i, c, 0, 0)),
            pl.BlockSpec(memory_space=pltpu.SMEM),
        ],
        out_specs=pl.BlockSpec((1, 1, H, W), lambda c, i: (c, i, 0, 0)),
        out_shape=jax.ShapeDtypeStruct((2, b, H, W), jnp.int32),
    )(pred5, target, class_weights)

    mesh = plsc.VectorSubcoreMesh(core_axis_name="c", subcore_axis_name="s")
    sc = functools.partial(
        pl.kernel,
        out_type=[
            jax.ShapeDtypeStruct((32, NLANE), jnp.float32),
            jax.ShapeDtypeStruct((32, NLANE), jnp.int32),
            jax.ShapeDtypeStruct((32, NLANE), jnp.int32),
            jax.ShapeDtypeStruct((32, NB), jnp.int32),
            jax.ShapeDtypeStruct((32, NLANE), jnp.float32),
        ],
        mesh=mesh,
        compiler_params=pltpu.CompilerParams(needs_layout_passes=False),
        scratch_types=[
            pltpu.VMEM((CHUNK,), jnp.int32),
            pltpu.VMEM((NB,), jnp.int32),
            pltpu.VMEM((4, NB), jnp.int32),
            pltpu.VMEM((4, NLANE), jnp.float32),
            pltpu.VMEM((NLANE,), jnp.int32),
            pltpu.VMEM((NLANE,), jnp.float32),
        ],
    )(_sc_body)
    s_part, k4_part, thr_part, _, _ = sc(bits.reshape(-1))

    sums2 = pl.pallas_call(
        _ce2_body,
        grid=(b,),
        in_specs=[
            pl.BlockSpec((1, 1, 2, H, W), lambda i: (i, 2, 0, 0, 0)),
            pl.BlockSpec((1, 1, H, W), lambda i: (i, 2, 0, 0)),
            pl.BlockSpec(memory_space=pltpu.SMEM),
        ],
        out_specs=pl.BlockSpec((1, b), lambda i: (0, 0),
                               memory_space=pltpu.SMEM),
        out_shape=jax.ShapeDtypeStruct((1, b), jnp.float32),
    )(pred5, target, class_weights)

    S = s_part.reshape(8, 4, NLANE)[:, 0, 0]
    K4 = k4_part.reshape(8, 4, NLANE)[:, 0, 0].astype(jnp.float32)
    T = lax.bitcast_convert_type(thr_part.reshape(8, 4, NLANE)[:, 0, 0],
                                 jnp.float32)
    topk = (S + K4 * T).reshape(2, 4)
    total = (jnp.sum(topk[0]) / (b * K)
             + jnp.sum(topk[1]) / (b * K)
             + jnp.sum(sums2) / (b * NPIX))
    return total


# speedup vs baseline: 3.3287x; 1.0346x over previous
import functools
import jax
import jax.numpy as jnp
from jax import lax
from jax.experimental import pallas as pl
from jax.experimental.pallas import tpu as pltpu
from jax.experimental.pallas import tpu_sc as plsc

IGNORE = 255
H = W = 400
NPIX = H * W
K = NPIX // 4
CHUNK = NPIX // 4
NLANE = 16
INNER = CHUNK // NLANE
NB = 2048


def _loss_plane(pred_ref, tgt_ref, cw_ref, c):
    p0 = pred_ref[0, 0, 0]
    p1 = pred_ref[0, 0, 1]
    t = tgt_ref[0, 0]
    valid = t != IGNORE
    is1 = t == 1
    d = jnp.where(is1, p0 - p1, p1 - p0)
    sp = jnp.maximum(d, 0.0) + jnp.log(1.0 + jnp.exp(-jnp.abs(d)))
    w = jnp.where(is1, cw_ref[c, 1], cw_ref[c, 0])
    return jnp.where(valid, w * sp, 0.0)


def _ce_body(pred_ref, tgt_ref, cw_ref, bits_ref):
    loss = _loss_plane(pred_ref, tgt_ref, cw_ref, pl.program_id(0))
    bits_ref[0, 0] = lax.bitcast_convert_type(loss, jnp.int32)


def _ce2_body(pred_ref, tgt_ref, cw_ref, sums_ref):
    i = pl.program_id(0)
    loss = _loss_plane(pred_ref, tgt_ref, cw_ref, 2)
    sums_ref[0, i] = jnp.sum(loss)


_GDN = lax.GatherDimensionNumbers(
    offset_dims=(), collapsed_slice_dims=(0,), start_index_map=(0,))


def _lane_pick(x, lane):
    idx = jnp.full((NLANE,), lane, jnp.int32)
    return lax.gather(x, idx[:, None], _GDN, (1,),
                      mode=lax.GatherScatterMode.PROMISE_IN_BOUNDS)


def _bcast_total(x):
    return _lane_pick(plsc.cumsum(x), NLANE - 1)


def _suffix_incl(x):
    return lax.rev(plsc.cumsum(lax.rev(x, (0,))), (0,))


def _sc_body(bits_hbm, s_out, c_out, t_out, xc_cnt,
             vals, hcnt, pc, stage_i, stage_f):
    cid = lax.axis_index("c")
    sid = lax.axis_index("s")
    wid = cid * 16 + sid
    rbase = (sid // 4) * 4

    pltpu.sync_copy(bits_hbm.at[pl.ds(wid * CHUNK, CHUNK)], vals)

    lane = lax.iota(jnp.int32, NLANE)
    ones_i = jnp.full((NLANE,), 1, jnp.int32)
    zeros_i = jnp.zeros((NLANE,), jnp.int32)
    zeros_f = jnp.zeros((NLANE,), jnp.float32)

    def zero_hists(_j, _):
        hcnt[pl.ds(_j * NLANE, NLANE)] = zeros_i
        return 0

    def exchange_and_merge():
        pltpu.sync_copy(hcnt, xc_cnt.at[wid])
        plsc.subcore_barrier()
        pltpu.sync_copy(xc_cnt.at[pl.ds(cid * 16 + rbase, 4)], pc)
        plsc.subcore_barrier()

    def scan_hist(nb, k_cur):
        nbv = nb // NLANE

        def body(_j, carry):
            cum_c, b_acc, a_acc = carry
            jj = nbv - 1 - _j
            d = pl.ds(jj * NLANE, NLANE)
            c = pc[0, d] + pc[1, d] + pc[2, d] + pc[3, d]
            sc_ = _suffix_incl(c)
            incl = cum_c + sc_
            excl = incl - c
            ind = ((excl < k_cur) & (incl >= k_cur)).astype(jnp.int32)
            b_acc = b_acc + ind * (jj * NLANE + lane)
            a_acc = a_acc + ind * excl
            cum_c = cum_c + _lane_pick(sc_, 0)
            return cum_c, b_acc, a_acc

        _, b, a = lax.fori_loop(
            0, nbv, body, (zeros_i, zeros_i, zeros_i))
        return _bcast_total(b), _bcast_total(a)

    lax.fori_loop(0, NB // NLANE, zero_hists, 0, unroll=4)

    def scat1(_j, _):
        d = pl.ds(_j * NLANE, NLANE)
        v = vals[d]
        idx = v >> 20
        plsc.addupdate_scatter(hcnt, [idx], ones_i)
        return 0
    lax.fori_loop(0, INNER, scat1, 0, unroll=8)
    exchange_and_merge()
    k1 = jnp.full((NLANE,), K, jnp.int32)
    b1, a1 = scan_hist(NB, k1)
    k2 = k1 - a1

    lax.fori_loop(0, NB // NLANE, zero_hists, 0, unroll=4)

    def scat2(_j, _):
        d = pl.ds(_j * NLANE, NLANE)
        v = vals[d]
        m = (v >> 20) == b1
        idx = (v >> 9) & (NB - 1)
        plsc.addupdate_scatter(hcnt, [idx], ones_i, mask=m)
        return 0
    lax.fori_loop(0, INNER, scat2, 0, unroll=8)
    exchange_and_merge()
    b2, a2 = scan_hist(NB, k2)
    k3 = k2 - a2

    lax.fori_loop(0, NB // NLANE, zero_hists, 0, unroll=4)
    pref = b1 * 2048 + b2

    def scat3(_j, _):
        d = pl.ds(_j * NLANE, NLANE)
        v = vals[d]
        m = (v >> 9) == pref
        idx = v & 511
        plsc.addupdate_scatter(hcnt, [idx], ones_i, mask=m)
        return 0
    lax.fori_loop(0, INNER, scat3, 0, unroll=8)
    exchange_and_merge()
    b3, a3 = scan_hist(512, k3)
    k4 = k3 - a3

    thr = b1 * 1048576 + b2 * 512 + b3

    def fsum(_j, acc):
        v = vals[pl.ds(_j * NLANE, NLANE)]
        return acc + jnp.where(v > thr, plsc.bitcast(v, jnp.float32), 0.0)
    s_above = lax.fori_loop(0, INNER, fsum, zeros_f, unroll=8)

    stage_i[...] = thr
    pltpu.sync_copy(stage_i, t_out.at[wid])
    stage_i[...] = k4
    pltpu.sync_copy(stage_i, c_out.at[wid])
    stage_f[...] = s_above
    pltpu.sync_copy(stage_f, s_out.at[wid])


@jax.jit
def kernel(prediction, target, class_weights):
    b = prediction.shape[0]
    pred5 = prediction.reshape(b, 3, 2, H, W)
    bits = pl.pallas_call(
        _ce_body,
        grid=(2, b),
        in_specs=[
            pl.BlockSpec((1, 1, 2, H, W), lambda c, i: (i, c, 0, 0, 0)),
            pl.BlockSpec((1, 1, H, W), lambda c, i: (i, c, 0, 0)),
            pl.BlockSpec(memory_space=pltpu.SMEM),
        ],
        out_specs=pl.BlockSpec((1, 1, H, W), lambda c, i: (c, i, 0, 0)),
        out_shape=jax.ShapeDtypeStruct((2, b, H, W), jnp.int32),
    )(pred5, target, class_weights)

    mesh = plsc.VectorSubcoreMesh(core_axis_name="c", subcore_axis_name="s")
    sc = functools.partial(
        pl.kernel,
        out_type=[
            jax.ShapeDtypeStruct((32, NLANE), jnp.float32),
            jax.ShapeDtypeStruct((32, NLANE), jnp.int32),
            jax.ShapeDtypeStruct((32, NLANE), jnp.int32),
            jax.ShapeDtypeStruct((32, NB), jnp.int32),
        ],
        mesh=mesh,
        compiler_params=pltpu.CompilerParams(needs_layout_passes=False),
        scratch_types=[
            pltpu.VMEM((CHUNK,), jnp.int32),
            pltpu.VMEM((NB,), jnp.int32),
            pltpu.VMEM((4, NB), jnp.int32),
            pltpu.VMEM((NLANE,), jnp.int32),
            pltpu.VMEM((NLANE,), jnp.float32),
        ],
    )(_sc_body)
    s_part, k4_part, thr_part, _ = sc(bits.reshape(-1))

    sums2 = pl.pallas_call(
        _ce2_body,
        grid=(b,),
        in_specs=[
            pl.BlockSpec((1, 1, 2, H, W), lambda i: (i, 2, 0, 0, 0)),
            pl.BlockSpec((1, 1, H, W), lambda i: (i, 2, 0, 0)),
            pl.BlockSpec(memory_space=pltpu.SMEM),
        ],
        out_specs=pl.BlockSpec((1, b), lambda i: (0, 0),
                               memory_space=pltpu.SMEM),
        out_shape=jax.ShapeDtypeStruct((1, b), jnp.float32),
    )(pred5, target, class_weights)

    S = jnp.sum(s_part.reshape(8, 4, NLANE), axis=(1, 2))
    K4 = k4_part.reshape(8, 4, NLANE)[:, 0, 0].astype(jnp.float32)
    T = lax.bitcast_convert_type(thr_part.reshape(8, 4, NLANE)[:, 0, 0],
                                 jnp.float32)
    topk = (S + K4 * T).reshape(2, 4)
    total = (jnp.sum(topk[0]) / (b * K)
             + jnp.sum(topk[1]) / (b * K)
             + jnp.sum(sums2) / (b * NPIX))
    return total
